# Initial kernel scaffold; baseline (speedup 1.0000x reference)
import jax
import jax.numpy as jnp
from jax.experimental import pallas as pl


def kernel(x, edge_index, batch, W_rel1, W_root1, b1, W_rel2, W_root2, b2, ln_g, ln_b, bn_g, bn_b, W_l1, b_l1, W_l2, b_l2, W_l3, b_l3):
    raise NotImplementedError("write your pallas kernel here")


# speedup vs baseline: 2.2926x; 2.2926x over previous
import functools

import jax
import jax.numpy as jnp
from jax import lax
from jax.experimental import pallas as pl
from jax.experimental.pallas import tpu as pltpu
from jax.experimental.pallas import tpu_sc as plsc

N = 10000
E = 160000
D = 256
DH = 128
NG = 64
EPS = 1e-5
NSUB = 16
EPT = E // NSUB
CH = 128
NFULL = EPT // CH
REM = EPT - NFULL * CH
NP = 10240
NH = NP // 2
ACCR = NH + 128
ZPT = ACCR // NSUB
RPT = NH // NSUB


def _route(idx_d, n, lo):
    for g in range(n // 16):
        v = idx_d[pl.ds(g * 16, 16)]
        rel = v - lo
        ok = (rel >= 0) & (rel < NH)
        idx_d[pl.ds(g * 16, 16)] = jnp.where(ok, rel, NH)


def _sc_body(h_lo, h_hi, src, dst, zeros, agg_lo, agg_hi,
             acc, idx_s, idx_d, rows, idx_s2, idx_d2, rows2, buf, sem):
    cid = lax.axis_index("c")
    sid = lax.axis_index("s")
    base0 = sid * EPT

    for p in range(2):
        lo = p * NH

        pltpu.sync_copy(zeros, buf)
        pltpu.sync_copy(buf, acc.at[pl.ds(sid * ZPT, ZPT)])
        plsc.subcore_barrier()

        def chunk(j, _):
            base = base0 + j * CH
            pltpu.sync_copy(src.at[pl.ds(base, CH)], idx_s)
            pltpu.sync_copy(dst.at[pl.ds(base, CH)], idx_d)
            _route(idx_d, CH, lo)

            @pl.when(cid == 0)
            def _():
                pltpu.async_copy(h_lo.at[idx_s], rows, sem).wait()

            @pl.when(cid == 1)
            def _():
                pltpu.async_copy(h_hi.at[idx_s], rows, sem).wait()

            pltpu.sync_copy(rows, acc.at[idx_d], add=True)
            return 0

        lax.fori_loop(0, NFULL, chunk, 0)

        base = base0 + NFULL * CH
        pltpu.sync_copy(src.at[pl.ds(base, REM)], idx_s2)
        pltpu.sync_copy(dst.at[pl.ds(base, REM)], idx_d2)
        _route(idx_d2, REM, lo)

        @pl.when(cid == 0)
        def _():
            pltpu.async_copy(h_lo.at[idx_s2], rows2, sem).wait()

        @pl.when(cid == 1)
        def _():
            pltpu.async_copy(h_hi.at[idx_s2], rows2, sem).wait()

        pltpu.sync_copy(rows2, acc.at[idx_d2], add=True)

        plsc.subcore_barrier()

        pltpu.sync_copy(acc.at[pl.ds(sid * RPT, RPT)], buf.at[pl.ds(0, RPT)])

        @pl.when(cid == 0)
        def _():
            pltpu.sync_copy(buf.at[pl.ds(0, RPT)],
                            agg_lo.at[pl.ds(lo + sid * RPT, RPT)])

        @pl.when(cid == 1)
        def _():
            pltpu.sync_copy(buf.at[pl.ds(0, RPT)],
                            agg_hi.at[pl.ds(lo + sid * RPT, RPT)])

        plsc.subcore_barrier()


def _sc_segsum(h_lo, h_hi, src, dst, zeros):
    mesh = plsc.VectorSubcoreMesh(core_axis_name="c", subcore_axis_name="s")
    f = pl.kernel(
        _sc_body,
        out_type=[jax.ShapeDtypeStruct((NP, DH), jnp.float32),
                  jax.ShapeDtypeStruct((NP, DH), jnp.float32)],
        mesh=mesh,
        scratch_types=[
            pltpu.VMEM_SHARED((ACCR, DH), jnp.float32),
            pltpu.VMEM((CH,), jnp.int32),
            pltpu.VMEM((CH,), jnp.int32),
            pltpu.VMEM((CH, DH), jnp.float32),
            pltpu.VMEM((REM,), jnp.int32),
            pltpu.VMEM((REM,), jnp.int32),
            pltpu.VMEM((REM, DH), jnp.float32),
            pltpu.VMEM((ZPT, DH), jnp.float32),
            pltpu.SemaphoreType.DMA,
        ],
    )
    return f(h_lo, h_hi, src, dst, zeros)


_RB = 1000


def _layer_body(agg_lo, agg_hi, h_lo, h_hi, wrel, wroot, bias, g, b,
                out_lo, out_hi):
    t = jnp.dot(agg_lo[...], wrel[:DH, :], preferred_element_type=jnp.float32)
    t += jnp.dot(agg_hi[...], wrel[DH:, :], preferred_element_type=jnp.float32)
    t += jnp.dot(h_lo[...], wroot[:DH, :], preferred_element_type=jnp.float32)
    t += jnp.dot(h_hi[...], wroot[DH:, :], preferred_element_type=jnp.float32)
    t += bias[...]
    m = jnp.mean(t, axis=1, keepdims=True)
    d = t - m
    v = jnp.mean(d * d, axis=1, keepdims=True)
    y = d / jnp.sqrt(v + EPS) * g[...] + b[...]
    out_lo[...] = y[:, :DH]
    out_hi[...] = y[:, DH:]


def _tc_layer(agg_lo, agg_hi, h_lo, h_hi, wrel, wroot, bias, g, b):
    grid = (N // _RB,)
    row = pl.BlockSpec((_RB, DH), lambda i: (i, 0))
    full = pl.BlockSpec((D, D), lambda i: (0, 0))
    vec = pl.BlockSpec((1, D), lambda i: (0, 0))
    return pl.pallas_call(
        _layer_body,
        grid=grid,
        in_specs=[row, row, row, row, full, full, vec, vec, vec],
        out_specs=[row, row],
        out_shape=[jax.ShapeDtypeStruct((N, DH), jnp.float32),
                   jax.ShapeDtypeStruct((N, DH), jnp.float32)],
    )(agg_lo, agg_hi, h_lo, h_hi, wrel, wroot, bias, g, b)


_PC = 16


def _pool_body(starts, h_lo, h_hi, out):
    gidx = pl.program_id(0)
    s = starts[gidx]
    e = starts[gidx + 1]
    a0 = (s // _PC) * _PC
    nt = (e - a0 + _PC - 1) // _PC

    neg = jnp.float32(-3.0e38)
    init = (jnp.zeros((_PC, DH), jnp.float32), jnp.zeros((_PC, DH), jnp.float32),
            jnp.full((_PC, DH), neg), jnp.full((_PC, DH), neg))

    def body(t, carry):
        s_lo, s_hi, m_lo, m_hi = carry
        r0 = a0 + t * _PC
        rl = h_lo[pl.ds(r0, _PC), :]
        rh = h_hi[pl.ds(r0, _PC), :]
        ridx = r0 + lax.broadcasted_iota(jnp.int32, (_PC, 1), 0)
        mask = (ridx >= s) & (ridx < e)
        s_lo = s_lo + jnp.where(mask, rl, 0.0)
        s_hi = s_hi + jnp.where(mask, rh, 0.0)
        m_lo = jnp.maximum(m_lo, jnp.where(mask, rl, neg))
        m_hi = jnp.maximum(m_hi, jnp.where(mask, rh, neg))
        return s_lo, s_hi, m_lo, m_hi

    s_lo, s_hi, m_lo, m_hi = lax.fori_loop(0, nt, body, init)

    cnt = (e - s).astype(jnp.float32)
    has = cnt > 0.0
    inv = 1.0 / jnp.maximum(cnt, 1.0)
    sum_l = jnp.sum(s_lo, axis=0, keepdims=True)
    sum_h = jnp.sum(s_hi, axis=0, keepdims=True)
    max_l = jnp.max(m_lo, axis=0, keepdims=True)
    max_h = jnp.max(m_hi, axis=0, keepdims=True)
    max_l = jnp.where(has, max_l, 0.0)
    max_h = jnp.where(has, max_h, 0.0)
    out[0, :, 0:DH] = sum_l * inv
    out[0, :, DH:D] = sum_h * inv
    out[0, :, D:D + DH] = max_l
    out[0, :, D + DH:2 * D] = max_h
    out[0, :, 2 * D:2 * D + DH] = sum_l
    out[0, :, 2 * D + DH:3 * D] = sum_h


def _tc_pool(starts, h_lo, h_hi):
    grid_spec = pltpu.PrefetchScalarGridSpec(
        num_scalar_prefetch=1,
        grid=(NG,),
        in_specs=[pl.BlockSpec((N, DH), lambda i, st: (0, 0)),
                  pl.BlockSpec((N, DH), lambda i, st: (0, 0))],
        out_specs=pl.BlockSpec((1, 1, 3 * D), lambda i, st: (i, 0, 0)),
    )
    res = pl.pallas_call(
        _pool_body,
        grid_spec=grid_spec,
        out_shape=jax.ShapeDtypeStruct((NG, 1, 3 * D), jnp.float32),
    )(starts, h_lo, h_hi)
    return res.reshape(NG, 3 * D)


def _head_body(hk, bn_g, bn_b, w1, b1, w2, b2, w3, b3, out):
    h = hk[...]
    bm = jnp.mean(h, axis=0, keepdims=True)
    d = h - bm
    bv = jnp.mean(d * d, axis=0, keepdims=True)
    xn = d / jnp.sqrt(bv + EPS) * bn_g[...] + bn_b[...]
    x1 = jnp.maximum(jnp.dot(xn, w1[...], preferred_element_type=jnp.float32)
                     + b1[...], 0.0)
    x2 = jnp.maximum(jnp.dot(x1, w2[...], preferred_element_type=jnp.float32)
                     + b2[...], 0.0)
    lg = jnp.dot(x2, w3[...], preferred_element_type=jnp.float32) + b3[...]
    col = lax.broadcasted_iota(jnp.int32, lg.shape, 1)
    valid = col < 2
    lgm = jnp.where(valid, lg, jnp.float32(-3.0e38))
    mx = jnp.max(lgm, axis=1, keepdims=True)
    ls = lgm - mx
    se = jnp.sum(jnp.where(valid, jnp.exp(ls), 0.0), axis=1, keepdims=True)
    out[...] = ls - jnp.log(se)


def _tc_head(hk, bn_g, bn_b, w1, b1, w2, b2, w3p, b3p):
    dcat = 3 * D * 3
    return pl.pallas_call(
        _head_body,
        out_shape=jax.ShapeDtypeStruct((NG, 128), jnp.float32),
    )(hk, bn_g.reshape(1, dcat), bn_b.reshape(1, dcat),
      w1, b1.reshape(1, -1), w2, b2.reshape(1, -1), w3p, b3p)


def kernel(x, edge_index, batch, W_rel1, W_root1, b1, W_rel2, W_root2, b2,
           ln_g, ln_b, bn_g, bn_b, W_l1, b_l1, W_l2, b_l2, W_l3, b_l3):
    src = edge_index[0]
    dst = edge_index[1]
    starts = jnp.searchsorted(
        batch, jnp.arange(NG + 1, dtype=jnp.int32)).astype(jnp.int32)
    zeros = jnp.zeros((ZPT, DH), jnp.float32)

    h_lo = x[:, :DH]
    h_hi = x[:, DH:]
    gv = ln_g.reshape(1, D)
    bv = ln_b.reshape(1, D)

    pools = []
    for k in range(3):
        wrel, wroot, bias = ((W_rel1, W_root1, b1) if k == 0
                             else (W_rel2, W_root2, b2))
        agg_lo, agg_hi = _sc_segsum(h_lo, h_hi, src, dst, zeros)
        h_lo, h_hi = _tc_layer(agg_lo, agg_hi, h_lo, h_hi,
                               wrel, wroot, bias.reshape(1, D), gv, bv)
        pools.append(_tc_pool(starts, h_lo, h_hi))

    hk = jnp.concatenate(pools, axis=1)
    w3p = jnp.pad(W_l3, ((0, 0), (0, 128 - W_l3.shape[1])))
    b3p = jnp.pad(b_l3, (0, 128 - b_l3.shape[0])).reshape(1, 128)
    out = _tc_head(hk, bn_g, bn_b, W_l1, b_l1, W_l2, b_l2, w3p, b3p)
    return out[:, :W_l3.shape[1]]


# speedup vs baseline: 3.3003x; 1.4395x over previous
import functools

import jax
import jax.numpy as jnp
from jax import lax
from jax.experimental import pallas as pl
from jax.experimental.pallas import tpu as pltpu
from jax.experimental.pallas import tpu_sc as plsc

N = 10000
E = 160000
D = 256
DH = 128
NG = 64
EPS = 1e-5
NSUB = 16
NW = 2 * NSUB
CH = 128
NP = 10240
NH = NP // 2
ACCR = NH + 128
ZPT = ACCR // NSUB
RPT = NH // NSUB
EPW = E // NW
NGRP = EPW // 16
BROW = 40
BCAP = BROW * CH


def _splat(val):
    return jnp.zeros((16,), jnp.int32) + val


def _gather16(x, idx):
    dnums = lax.GatherDimensionNumbers(
        offset_dims=(), collapsed_slice_dims=(0,), start_index_map=(0,))
    return lax.gather(x, idx[:, None], dnums, slice_sizes=(1,),
                      mode=lax.GatherScatterMode.PROMISE_IN_BOUNDS)


def _cumsum16(x):
    lane = lax.broadcasted_iota(jnp.int32, (16,), 0)
    for k in (1, 2, 4, 8):
        sh = _gather16(x, jnp.maximum(lane - k, 0))
        x = x + jnp.where(lane >= k, sh, 0)
    return x


def _compact16(m, vals):
    lane = lax.broadcasted_iota(jnp.int32, (16,), 0)
    cum = _cumsum16(jnp.where(m, 1, 0))
    c = cum[15]
    i1 = lane + 1
    pos = jnp.zeros((16,), jnp.int32)
    for s in (8, 4, 2, 1):
        nxt = pos + s
        cmid = _gather16(cum, nxt - 1)
        pos = jnp.where(cmid < i1, nxt, pos)
    pos = jnp.minimum(pos, 15)
    outm = lane < c
    return c, outm, [_gather16(v, pos) for v in vals]


def _route_body(src, dst, zeros_i, trash_i, s0, d0, s1, d1, counts,
                srcin, dstin, bs0, bd0, bs1, bd1, cbuf):
    cid = lax.axis_index("c")
    sid = lax.axis_index("s")
    wid = cid * NSUB + sid
    base = wid * EPW

    pltpu.sync_copy(src.at[pl.ds(base, EPW)], srcin)
    pltpu.sync_copy(dst.at[pl.ds(base, EPW)], dstin)
    pltpu.sync_copy(zeros_i, bs0)
    pltpu.sync_copy(zeros_i, bs1)
    pltpu.sync_copy(trash_i, bd0)
    pltpu.sync_copy(trash_i, bd1)

    def group(sv, dv, valid, off0, off1):
        if valid is None:
            m0 = dv < NH
            m1 = dv >= NH
        else:
            m0 = valid & (dv < NH)
            m1 = valid & (dv >= NH)
        c0, outm0, (sv0, dv0) = _compact16(m0, [sv, dv])
        bs0[pl.ds(off0, 16)] = jnp.where(outm0, sv0, 0)
        bd0[pl.ds(off0, 16)] = jnp.where(outm0, dv0, NH)
        c1, outm1, (sv1, dv1) = _compact16(m1, [sv, dv])
        bs1[pl.ds(off1, 16)] = jnp.where(outm1, sv1, 0)
        bd1[pl.ds(off1, 16)] = jnp.where(outm1, dv1 - NH, NH)
        return off0 + c0, off1 + c1

    def body(g, carry):
        off0, off1 = carry
        sv = srcin[pl.ds(g * 16, 16)]
        dv = dstin[pl.ds(g * 16, 16)]
        return group(sv, dv, None, off0, off1)

    off0, off1 = lax.fori_loop(0, NGRP, body, (0, 0))

    sv = srcin[pl.ds(EPW - 16, 16)]
    dv = dstin[pl.ds(EPW - 16, 16)]
    lane = lax.broadcasted_iota(jnp.int32, (16,), 0)
    off0, off1 = group(sv, dv, lane >= 8, off0, off1)

    for k in range(8):
        cbuf[0, pl.ds(k * 16, 16)] = _splat(off0)
        cbuf[1, pl.ds(k * 16, 16)] = _splat(off1)

    out0 = wid * BCAP
    pltpu.sync_copy(bs0, s0.at[pl.ds(out0, BCAP)])
    pltpu.sync_copy(bd0, d0.at[pl.ds(out0, BCAP)])
    pltpu.sync_copy(bs1, s1.at[pl.ds(out0, BCAP)])
    pltpu.sync_copy(bd1, d1.at[pl.ds(out0, BCAP)])
    pltpu.sync_copy(cbuf, counts.at[wid])


def _sc_route(src, dst, zeros_i, trash_i):
    mesh = plsc.VectorSubcoreMesh(core_axis_name="c", subcore_axis_name="s")
    bkt = jax.ShapeDtypeStruct((NW * BCAP,), jnp.int32)
    f = pl.kernel(
        _route_body,
        out_type=[bkt, bkt, bkt, bkt,
                  jax.ShapeDtypeStruct((NW, 8, CH), jnp.int32)],
        mesh=mesh,
        scratch_types=[
            pltpu.VMEM((EPW,), jnp.int32),
            pltpu.VMEM((EPW,), jnp.int32),
            pltpu.VMEM((BCAP,), jnp.int32),
            pltpu.VMEM((BCAP,), jnp.int32),
            pltpu.VMEM((BCAP,), jnp.int32),
            pltpu.VMEM((BCAP,), jnp.int32),
            pltpu.VMEM((8, CH), jnp.int32),
        ],
    )
    return f(src, dst, zeros_i, trash_i)


def _sc_body(h_lo, h_hi, s0, d0, s1, d1, counts, zeros, agg_lo, agg_hi,
             acc, vsrc, vdst, idx_s, idx_d, cbuf, rows, buf, sem):
    cid = lax.axis_index("c")
    sid = lax.axis_index("s")

    for p, (sb, db) in enumerate(((s0, d0), (s1, d1))):
        lo = p * NH

        pltpu.sync_copy(zeros, buf)
        pltpu.sync_copy(buf, acc.at[pl.ds(sid * ZPT, ZPT)])
        plsc.subcore_barrier()

        for rbo in range(2):
            rb = 2 * sid + rbo
            pltpu.sync_copy(sb.at[pl.ds(rb * BCAP, BCAP)], vsrc)
            pltpu.sync_copy(db.at[pl.ds(rb * BCAP, BCAP)], vdst)
            pltpu.sync_copy(counts.at[rb], cbuf)
            cnt = cbuf[p, pl.ds(0, 16)][0]
            nch = (cnt + CH - 1) // CH

            def chunk(j, _):
                for g in range(CH // 16):
                    idx_s[pl.ds(g * 16, 16)] = vsrc[pl.ds(j * CH + g * 16, 16)]
                    idx_d[pl.ds(g * 16, 16)] = vdst[pl.ds(j * CH + g * 16, 16)]

                @pl.when(cid == 0)
                def _():
                    pltpu.async_copy(h_lo.at[idx_s], rows, sem).wait()

                @pl.when(cid == 1)
                def _():
                    pltpu.async_copy(h_hi.at[idx_s], rows, sem).wait()

                pltpu.sync_copy(rows, acc.at[idx_d], add=True)
                return 0

            lax.fori_loop(0, nch, chunk, 0)

        plsc.subcore_barrier()

        pltpu.sync_copy(acc.at[pl.ds(sid * RPT, RPT)], buf.at[pl.ds(0, RPT)])

        @pl.when(cid == 0)
        def _():
            pltpu.sync_copy(buf.at[pl.ds(0, RPT)],
                            agg_lo.at[pl.ds(lo + sid * RPT, RPT)])

        @pl.when(cid == 1)
        def _():
            pltpu.sync_copy(buf.at[pl.ds(0, RPT)],
                            agg_hi.at[pl.ds(lo + sid * RPT, RPT)])

        plsc.subcore_barrier()


def _sc_segsum(h_lo, h_hi, s0, d0, s1, d1, counts, zeros):
    mesh = plsc.VectorSubcoreMesh(core_axis_name="c", subcore_axis_name="s")
    f = pl.kernel(
        _sc_body,
        out_type=[jax.ShapeDtypeStruct((NP, DH), jnp.float32),
                  jax.ShapeDtypeStruct((NP, DH), jnp.float32)],
        mesh=mesh,
        scratch_types=[
            pltpu.VMEM_SHARED((ACCR, DH), jnp.float32),
            pltpu.VMEM((BCAP,), jnp.int32),
            pltpu.VMEM((BCAP,), jnp.int32),
            pltpu.VMEM((CH,), jnp.int32),
            pltpu.VMEM((CH,), jnp.int32),
            pltpu.VMEM((8, CH), jnp.int32),
            pltpu.VMEM((CH, DH), jnp.float32),
            pltpu.VMEM((ZPT, DH), jnp.float32),
            pltpu.SemaphoreType.DMA,
        ],
    )
    return f(h_lo, h_hi, s0, d0, s1, d1, counts, zeros)


_RB = 1000


def _layer_body(agg_lo, agg_hi, h_lo, h_hi, wrel, wroot, bias, g, b,
                out_lo, out_hi):
    t = jnp.dot(agg_lo[...], wrel[:DH, :], preferred_element_type=jnp.float32)
    t += jnp.dot(agg_hi[...], wrel[DH:, :], preferred_element_type=jnp.float32)
    t += jnp.dot(h_lo[...], wroot[:DH, :], preferred_element_type=jnp.float32)
    t += jnp.dot(h_hi[...], wroot[DH:, :], preferred_element_type=jnp.float32)
    t += bias[...]
    m = jnp.mean(t, axis=1, keepdims=True)
    d = t - m
    v = jnp.mean(d * d, axis=1, keepdims=True)
    y = d / jnp.sqrt(v + EPS) * g[...] + b[...]
    out_lo[...] = y[:, :DH]
    out_hi[...] = y[:, DH:]


def _tc_layer(agg_lo, agg_hi, h_lo, h_hi, wrel, wroot, bias, g, b):
    grid = (N // _RB,)
    row = pl.BlockSpec((_RB, DH), lambda i: (i, 0))
    full = pl.BlockSpec((D, D), lambda i: (0, 0))
    vec = pl.BlockSpec((1, D), lambda i: (0, 0))
    return pl.pallas_call(
        _layer_body,
        grid=grid,
        in_specs=[row, row, row, row, full, full, vec, vec, vec],
        out_specs=[row, row],
        out_shape=[jax.ShapeDtypeStruct((N, DH), jnp.float32),
                   jax.ShapeDtypeStruct((N, DH), jnp.float32)],
    )(agg_lo, agg_hi, h_lo, h_hi, wrel, wroot, bias, g, b)


_PC = 16


def _pool_body(starts, h_lo, h_hi, out):
    gidx = pl.program_id(0)
    s = starts[gidx]
    e = starts[gidx + 1]
    a0 = (s // _PC) * _PC
    nt = (e - a0 + _PC - 1) // _PC

    neg = jnp.float32(-3.0e38)
    init = (jnp.zeros((_PC, DH), jnp.float32), jnp.zeros((_PC, DH), jnp.float32),
            jnp.full((_PC, DH), neg), jnp.full((_PC, DH), neg))

    def body(t, carry):
        s_lo, s_hi, m_lo, m_hi = carry
        r0 = a0 + t * _PC
        rl = h_lo[pl.ds(r0, _PC), :]
        rh = h_hi[pl.ds(r0, _PC), :]
        ridx = r0 + lax.broadcasted_iota(jnp.int32, (_PC, 1), 0)
        mask = (ridx >= s) & (ridx < e)
        s_lo = s_lo + jnp.where(mask, rl, 0.0)
        s_hi = s_hi + jnp.where(mask, rh, 0.0)
        m_lo = jnp.maximum(m_lo, jnp.where(mask, rl, neg))
        m_hi = jnp.maximum(m_hi, jnp.where(mask, rh, neg))
        return s_lo, s_hi, m_lo, m_hi

    s_lo, s_hi, m_lo, m_hi = lax.fori_loop(0, nt, body, init)

    cnt = (e - s).astype(jnp.float32)
    has = cnt > 0.0
    inv = 1.0 / jnp.maximum(cnt, 1.0)
    sum_l = jnp.sum(s_lo, axis=0, keepdims=True)
    sum_h = jnp.sum(s_hi, axis=0, keepdims=True)
    max_l = jnp.max(m_lo, axis=0, keepdims=True)
    max_h = jnp.max(m_hi, axis=0, keepdims=True)
    max_l = jnp.where(has, max_l, 0.0)
    max_h = jnp.where(has, max_h, 0.0)
    out[0, :, 0:DH] = sum_l * inv
    out[0, :, DH:D] = sum_h * inv
    out[0, :, D:D + DH] = max_l
    out[0, :, D + DH:2 * D] = max_h
    out[0, :, 2 * D:2 * D + DH] = sum_l
    out[0, :, 2 * D + DH:3 * D] = sum_h


def _tc_pool(starts, h_lo, h_hi):
    grid_spec = pltpu.PrefetchScalarGridSpec(
        num_scalar_prefetch=1,
        grid=(NG,),
        in_specs=[pl.BlockSpec((N, DH), lambda i, st: (0, 0)),
                  pl.BlockSpec((N, DH), lambda i, st: (0, 0))],
        out_specs=pl.BlockSpec((1, 1, 3 * D), lambda i, st: (i, 0, 0)),
    )
    res = pl.pallas_call(
        _pool_body,
        grid_spec=grid_spec,
        out_shape=jax.ShapeDtypeStruct((NG, 1, 3 * D), jnp.float32),
    )(starts, h_lo, h_hi)
    return res.reshape(NG, 3 * D)


def _head_body(hk, bn_g, bn_b, w1, b1, w2, b2, w3, b3, out):
    h = hk[...]
    bm = jnp.mean(h, axis=0, keepdims=True)
    d = h - bm
    bv = jnp.mean(d * d, axis=0, keepdims=True)
    xn = d / jnp.sqrt(bv + EPS) * bn_g[...] + bn_b[...]
    x1 = jnp.maximum(jnp.dot(xn, w1[...], preferred_element_type=jnp.float32)
                     + b1[...], 0.0)
    x2 = jnp.maximum(jnp.dot(x1, w2[...], preferred_element_type=jnp.float32)
                     + b2[...], 0.0)
    lg = jnp.dot(x2, w3[...], preferred_element_type=jnp.float32) + b3[...]
    col = lax.broadcasted_iota(jnp.int32, lg.shape, 1)
    valid = col < 2
    lgm = jnp.where(valid, lg, jnp.float32(-3.0e38))
    mx = jnp.max(lgm, axis=1, keepdims=True)
    ls = lgm - mx
    se = jnp.sum(jnp.where(valid, jnp.exp(ls), 0.0), axis=1, keepdims=True)
    out[...] = ls - jnp.log(se)


def _tc_head(hk, bn_g, bn_b, w1, b1, w2, b2, w3p, b3p):
    dcat = 3 * D * 3
    return pl.pallas_call(
        _head_body,
        out_shape=jax.ShapeDtypeStruct((NG, 128), jnp.float32),
    )(hk, bn_g.reshape(1, dcat), bn_b.reshape(1, dcat),
      w1, b1.reshape(1, -1), w2, b2.reshape(1, -1), w3p, b3p)


def kernel(x, edge_index, batch, W_rel1, W_root1, b1, W_rel2, W_root2, b2,
           ln_g, ln_b, bn_g, bn_b, W_l1, b_l1, W_l2, b_l2, W_l3, b_l3):
    src = edge_index[0]
    dst = edge_index[1]
    starts = jnp.searchsorted(
        batch, jnp.arange(NG + 1, dtype=jnp.int32)).astype(jnp.int32)
    zeros = jnp.zeros((ZPT, DH), jnp.float32)
    zeros_i = jnp.zeros((BCAP,), jnp.int32)
    trash_i = jnp.zeros((BCAP,), jnp.int32) + NH

    s0, d0, s1, d1, counts = _sc_route(src, dst, zeros_i, trash_i)

    h_lo = x[:, :DH]
    h_hi = x[:, DH:]
    gv = ln_g.reshape(1, D)
    bv = ln_b.reshape(1, D)

    pools = []
    for k in range(3):
        wrel, wroot, bias = ((W_rel1, W_root1, b1) if k == 0
                             else (W_rel2, W_root2, b2))
        agg_lo, agg_hi = _sc_segsum(h_lo, h_hi, s0, d0, s1, d1, counts, zeros)
        h_lo, h_hi = _tc_layer(agg_lo, agg_hi, h_lo, h_hi,
                               wrel, wroot, bias.reshape(1, D), gv, bv)
        pools.append(_tc_pool(starts, h_lo, h_hi))

    hk = jnp.concatenate(pools, axis=1)
    w3p = jnp.pad(W_l3, ((0, 0), (0, 128 - W_l3.shape[1])))
    b3p = jnp.pad(b_l3, (0, 128 - b_l3.shape[0])).reshape(1, 128)
    out = _tc_head(hk, bn_g, bn_b, W_l1, b_l1, W_l2, b_l2, w3p, b3p)
    return out[:, :W_l3.shape[1]]


# speedup vs baseline: 3.6256x; 1.0986x over previous
import functools

import jax
import jax.numpy as jnp
from jax import lax
from jax.experimental import pallas as pl
from jax.experimental.pallas import tpu as pltpu
from jax.experimental.pallas import tpu_sc as plsc

N = 10000
E = 160000
D = 256
DH = 128
NG = 64
EPS = 1e-5
NSUB = 16
NW = 2 * NSUB
CH = 128
NP = 10240
NH = NP // 2
ACCR = NH + 128
ZPT = ACCR // NSUB
RPT = NH // NSUB
EPW = E // NW
NGRP = EPW // 16
BROW = 40
BCAP = BROW * CH


def _splat(val):
    return jnp.zeros((16,), jnp.int32) + val


def _gather16(x, idx):
    dnums = lax.GatherDimensionNumbers(
        offset_dims=(), collapsed_slice_dims=(0,), start_index_map=(0,))
    return lax.gather(x, idx[:, None], dnums, slice_sizes=(1,),
                      mode=lax.GatherScatterMode.PROMISE_IN_BOUNDS)


def _cumsum16(x):
    lane = lax.broadcasted_iota(jnp.int32, (16,), 0)
    for k in (1, 2, 4, 8):
        sh = _gather16(x, jnp.maximum(lane - k, 0))
        x = x + jnp.where(lane >= k, sh, 0)
    return x


def _compact16(m, vals):
    lane = lax.broadcasted_iota(jnp.int32, (16,), 0)
    cum = _cumsum16(jnp.where(m, 1, 0))
    c = cum[15]
    i1 = lane + 1
    pos = jnp.zeros((16,), jnp.int32)
    for s in (8, 4, 2, 1):
        nxt = pos + s
        cmid = _gather16(cum, nxt - 1)
        pos = jnp.where(cmid < i1, nxt, pos)
    pos = jnp.minimum(pos, 15)
    outm = lane < c
    return c, outm, [_gather16(v, pos) for v in vals]


def _route_body(src, dst, zeros_i, trash_i, s0, d0, s1, d1, counts,
                srcin, dstin, bs0, bd0, bs1, bd1, cbuf):
    cid = lax.axis_index("c")
    sid = lax.axis_index("s")
    wid = cid * NSUB + sid
    base = wid * EPW

    pltpu.sync_copy(src.at[pl.ds(base, EPW)], srcin)
    pltpu.sync_copy(dst.at[pl.ds(base, EPW)], dstin)
    pltpu.sync_copy(zeros_i, bs0)
    pltpu.sync_copy(zeros_i, bs1)
    pltpu.sync_copy(trash_i, bd0)
    pltpu.sync_copy(trash_i, bd1)

    def group(sv, dv, valid, off0, off1):
        if valid is None:
            m0 = dv < NH
            m1 = dv >= NH
        else:
            m0 = valid & (dv < NH)
            m1 = valid & (dv >= NH)
        c0, outm0, (sv0, dv0) = _compact16(m0, [sv, dv])
        bs0[pl.ds(off0, 16)] = jnp.where(outm0, sv0, 0)
        bd0[pl.ds(off0, 16)] = jnp.where(outm0, dv0, NH)
        c1, outm1, (sv1, dv1) = _compact16(m1, [sv, dv])
        bs1[pl.ds(off1, 16)] = jnp.where(outm1, sv1, 0)
        bd1[pl.ds(off1, 16)] = jnp.where(outm1, dv1 - NH, NH)
        return off0 + c0, off1 + c1

    def body(g, carry):
        off0, off1 = carry
        sv = srcin[pl.ds(g * 16, 16)]
        dv = dstin[pl.ds(g * 16, 16)]
        return group(sv, dv, None, off0, off1)

    off0, off1 = lax.fori_loop(0, NGRP, body, (0, 0))

    sv = srcin[pl.ds(EPW - 16, 16)]
    dv = dstin[pl.ds(EPW - 16, 16)]
    lane = lax.broadcasted_iota(jnp.int32, (16,), 0)
    off0, off1 = group(sv, dv, lane >= 8, off0, off1)

    for k in range(8):
        cbuf[0, pl.ds(k * 16, 16)] = _splat(off0)
        cbuf[1, pl.ds(k * 16, 16)] = _splat(off1)

    out0 = wid * BCAP
    pltpu.sync_copy(bs0, s0.at[pl.ds(out0, BCAP)])
    pltpu.sync_copy(bd0, d0.at[pl.ds(out0, BCAP)])
    pltpu.sync_copy(bs1, s1.at[pl.ds(out0, BCAP)])
    pltpu.sync_copy(bd1, d1.at[pl.ds(out0, BCAP)])
    pltpu.sync_copy(cbuf, counts.at[wid])


def _sc_route(src, dst, zeros_i, trash_i):
    mesh = plsc.VectorSubcoreMesh(core_axis_name="c", subcore_axis_name="s")
    bkt = jax.ShapeDtypeStruct((NW * BCAP,), jnp.int32)
    f = pl.kernel(
        _route_body,
        out_type=[bkt, bkt, bkt, bkt,
                  jax.ShapeDtypeStruct((NW, 8, CH), jnp.int32)],
        mesh=mesh,
        scratch_types=[
            pltpu.VMEM((EPW,), jnp.int32),
            pltpu.VMEM((EPW,), jnp.int32),
            pltpu.VMEM((BCAP,), jnp.int32),
            pltpu.VMEM((BCAP,), jnp.int32),
            pltpu.VMEM((BCAP,), jnp.int32),
            pltpu.VMEM((BCAP,), jnp.int32),
            pltpu.VMEM((8, CH), jnp.int32),
        ],
    )
    return f(src, dst, zeros_i, trash_i)


def _sc_body(h_lo, h_hi, s0, d0, s1, d1, counts, zeros, agg_lo, agg_hi,
             acc, vsrc, vdst, idx_s0, idx_d0, idx_s1, idx_d1, rows0, rows1,
             cbuf, buf, sem):
    cid = lax.axis_index("c")
    sid = lax.axis_index("s")
    idx_s = (idx_s0, idx_s1)
    idx_d = (idx_d0, idx_d1)
    rows = (rows0, rows1)

    def fill_idx(j, jb):
        for g in range(CH // 16):
            idx_s[jb][pl.ds(g * 16, 16)] = vsrc[pl.ds(j * CH + g * 16, 16)]
            idx_d[jb][pl.ds(g * 16, 16)] = vdst[pl.ds(j * CH + g * 16, 16)]

    def run_row(h_ref, nch):
        @pl.when(nch > 0)
        def _():
            fill_idx(0, 0)
            pltpu.async_copy(h_ref.at[idx_s[0]], rows[0], sem)

        def step(j, jb):
            pltpu.make_async_copy(h_ref.at[idx_s[jb]], rows[jb], sem).wait()

            @pl.when(j + 1 < nch)
            def _():
                fill_idx(j + 1, 1 - jb)
                pltpu.async_copy(h_ref.at[idx_s[1 - jb]], rows[1 - jb], sem)

            pltpu.sync_copy(rows[jb], acc.at[idx_d[jb]], add=True)

        def body(j, _):
            @pl.when(j % 2 == 0)
            def _():
                step(j, 0)

            @pl.when(j % 2 == 1)
            def _():
                step(j, 1)

            return 0

        lax.fori_loop(0, nch, body, 0)

    for p, (sb, db) in enumerate(((s0, d0), (s1, d1))):
        lo = p * NH

        pltpu.sync_copy(zeros, buf)
        pltpu.sync_copy(buf, acc.at[pl.ds(sid * ZPT, ZPT)])
        plsc.subcore_barrier()

        for rbo in range(2):
            rb = 2 * sid + rbo
            pltpu.sync_copy(sb.at[pl.ds(rb * BCAP, BCAP)], vsrc)
            pltpu.sync_copy(db.at[pl.ds(rb * BCAP, BCAP)], vdst)
            pltpu.sync_copy(counts.at[rb], cbuf)
            cnt = cbuf[p, pl.ds(0, 16)][0]
            nch = (cnt + CH - 1) // CH

            @pl.when(cid == 0)
            def _():
                run_row(h_lo, nch)

            @pl.when(cid == 1)
            def _():
                run_row(h_hi, nch)

        plsc.subcore_barrier()

        pltpu.sync_copy(acc.at[pl.ds(sid * RPT, RPT)], buf.at[pl.ds(0, RPT)])

        @pl.when(cid == 0)
        def _():
            pltpu.sync_copy(buf.at[pl.ds(0, RPT)],
                            agg_lo.at[pl.ds(lo + sid * RPT, RPT)])

        @pl.when(cid == 1)
        def _():
            pltpu.sync_copy(buf.at[pl.ds(0, RPT)],
                            agg_hi.at[pl.ds(lo + sid * RPT, RPT)])

        plsc.subcore_barrier()


def _sc_segsum(h_lo, h_hi, s0, d0, s1, d1, counts, zeros):
    mesh = plsc.VectorSubcoreMesh(core_axis_name="c", subcore_axis_name="s")
    f = pl.kernel(
        _sc_body,
        out_type=[jax.ShapeDtypeStruct((NP, DH), jnp.float32),
                  jax.ShapeDtypeStruct((NP, DH), jnp.float32)],
        mesh=mesh,
        scratch_types=[
            pltpu.VMEM_SHARED((ACCR, DH), jnp.float32),
            pltpu.VMEM((BCAP,), jnp.int32),
            pltpu.VMEM((BCAP,), jnp.int32),
            pltpu.VMEM((CH,), jnp.int32),
            pltpu.VMEM((CH,), jnp.int32),
            pltpu.VMEM((CH,), jnp.int32),
            pltpu.VMEM((CH,), jnp.int32),
            pltpu.VMEM((CH, DH), jnp.float32),
            pltpu.VMEM((CH, DH), jnp.float32),
            pltpu.VMEM((8, CH), jnp.int32),
            pltpu.VMEM((ZPT, DH), jnp.float32),
            pltpu.SemaphoreType.DMA,
        ],
    )
    return f(h_lo, h_hi, s0, d0, s1, d1, counts, zeros)


_RB = 1000


def _layer_body(agg_lo, agg_hi, h_lo, h_hi, wrel, wroot, bias, g, b,
                out_lo, out_hi):
    t = jnp.dot(agg_lo[...], wrel[:DH, :], preferred_element_type=jnp.float32)
    t += jnp.dot(agg_hi[...], wrel[DH:, :], preferred_element_type=jnp.float32)
    t += jnp.dot(h_lo[...], wroot[:DH, :], preferred_element_type=jnp.float32)
    t += jnp.dot(h_hi[...], wroot[DH:, :], preferred_element_type=jnp.float32)
    t += bias[...]
    m = jnp.mean(t, axis=1, keepdims=True)
    d = t - m
    v = jnp.mean(d * d, axis=1, keepdims=True)
    y = d / jnp.sqrt(v + EPS) * g[...] + b[...]
    out_lo[...] = y[:, :DH]
    out_hi[...] = y[:, DH:]


def _tc_layer(agg_lo, agg_hi, h_lo, h_hi, wrel, wroot, bias, g, b):
    grid = (N // _RB,)
    row = pl.BlockSpec((_RB, DH), lambda i: (i, 0))
    full = pl.BlockSpec((D, D), lambda i: (0, 0))
    vec = pl.BlockSpec((1, D), lambda i: (0, 0))
    return pl.pallas_call(
        _layer_body,
        grid=grid,
        in_specs=[row, row, row, row, full, full, vec, vec, vec],
        out_specs=[row, row],
        out_shape=[jax.ShapeDtypeStruct((N, DH), jnp.float32),
                   jax.ShapeDtypeStruct((N, DH), jnp.float32)],
    )(agg_lo, agg_hi, h_lo, h_hi, wrel, wroot, bias, g, b)


_PC = 16


def _pool_body(starts, h_lo, h_hi, out):
    gidx = pl.program_id(0)
    s = starts[gidx]
    e = starts[gidx + 1]
    a0 = (s // _PC) * _PC
    nt = (e - a0 + _PC - 1) // _PC

    neg = jnp.float32(-3.0e38)
    init = (jnp.zeros((_PC, DH), jnp.float32), jnp.zeros((_PC, DH), jnp.float32),
            jnp.full((_PC, DH), neg), jnp.full((_PC, DH), neg))

    def body(t, carry):
        s_lo, s_hi, m_lo, m_hi = carry
        r0 = a0 + t * _PC
        rl = h_lo[pl.ds(r0, _PC), :]
        rh = h_hi[pl.ds(r0, _PC), :]
        ridx = r0 + lax.broadcasted_iota(jnp.int32, (_PC, 1), 0)
        mask = (ridx >= s) & (ridx < e)
        s_lo = s_lo + jnp.where(mask, rl, 0.0)
        s_hi = s_hi + jnp.where(mask, rh, 0.0)
        m_lo = jnp.maximum(m_lo, jnp.where(mask, rl, neg))
        m_hi = jnp.maximum(m_hi, jnp.where(mask, rh, neg))
        return s_lo, s_hi, m_lo, m_hi

    s_lo, s_hi, m_lo, m_hi = lax.fori_loop(0, nt, body, init)

    cnt = (e - s).astype(jnp.float32)
    has = cnt > 0.0
    inv = 1.0 / jnp.maximum(cnt, 1.0)
    sum_l = jnp.sum(s_lo, axis=0, keepdims=True)
    sum_h = jnp.sum(s_hi, axis=0, keepdims=True)
    max_l = jnp.max(m_lo, axis=0, keepdims=True)
    max_h = jnp.max(m_hi, axis=0, keepdims=True)
    max_l = jnp.where(has, max_l, 0.0)
    max_h = jnp.where(has, max_h, 0.0)
    out[0, :, 0:DH] = sum_l * inv
    out[0, :, DH:D] = sum_h * inv
    out[0, :, D:D + DH] = max_l
    out[0, :, D + DH:2 * D] = max_h
    out[0, :, 2 * D:2 * D + DH] = sum_l
    out[0, :, 2 * D + DH:3 * D] = sum_h


def _tc_pool(starts, h_lo, h_hi):
    grid_spec = pltpu.PrefetchScalarGridSpec(
        num_scalar_prefetch=1,
        grid=(NG,),
        in_specs=[pl.BlockSpec((N, DH), lambda i, st: (0, 0)),
                  pl.BlockSpec((N, DH), lambda i, st: (0, 0))],
        out_specs=pl.BlockSpec((1, 1, 3 * D), lambda i, st: (i, 0, 0)),
    )
    res = pl.pallas_call(
        _pool_body,
        grid_spec=grid_spec,
        out_shape=jax.ShapeDtypeStruct((NG, 1, 3 * D), jnp.float32),
    )(starts, h_lo, h_hi)
    return res.reshape(NG, 3 * D)


def _head_body(hk, bn_g, bn_b, w1, b1, w2, b2, w3, b3, out):
    h = hk[...]
    bm = jnp.mean(h, axis=0, keepdims=True)
    d = h - bm
    bv = jnp.mean(d * d, axis=0, keepdims=True)
    xn = d / jnp.sqrt(bv + EPS) * bn_g[...] + bn_b[...]
    x1 = jnp.maximum(jnp.dot(xn, w1[...], preferred_element_type=jnp.float32)
                     + b1[...], 0.0)
    x2 = jnp.maximum(jnp.dot(x1, w2[...], preferred_element_type=jnp.float32)
                     + b2[...], 0.0)
    lg = jnp.dot(x2, w3[...], preferred_element_type=jnp.float32) + b3[...]
    col = lax.broadcasted_iota(jnp.int32, lg.shape, 1)
    valid = col < 2
    lgm = jnp.where(valid, lg, jnp.float32(-3.0e38))
    mx = jnp.max(lgm, axis=1, keepdims=True)
    ls = lgm - mx
    se = jnp.sum(jnp.where(valid, jnp.exp(ls), 0.0), axis=1, keepdims=True)
    out[...] = ls - jnp.log(se)


def _tc_head(hk, bn_g, bn_b, w1, b1, w2, b2, w3p, b3p):
    dcat = 3 * D * 3
    return pl.pallas_call(
        _head_body,
        out_shape=jax.ShapeDtypeStruct((NG, 128), jnp.float32),
    )(hk, bn_g.reshape(1, dcat), bn_b.reshape(1, dcat),
      w1, b1.reshape(1, -1), w2, b2.reshape(1, -1), w3p, b3p)


def kernel(x, edge_index, batch, W_rel1, W_root1, b1, W_rel2, W_root2, b2,
           ln_g, ln_b, bn_g, bn_b, W_l1, b_l1, W_l2, b_l2, W_l3, b_l3):
    src = edge_index[0]
    dst = edge_index[1]
    starts = jnp.searchsorted(
        batch, jnp.arange(NG + 1, dtype=jnp.int32)).astype(jnp.int32)
    zeros = jnp.zeros((ZPT, DH), jnp.float32)
    zeros_i = jnp.zeros((BCAP,), jnp.int32)
    trash_i = jnp.zeros((BCAP,), jnp.int32) + NH

    s0, d0, s1, d1, counts = _sc_route(src, dst, zeros_i, trash_i)

    h_lo = x[:, :DH]
    h_hi = x[:, DH:]
    gv = ln_g.reshape(1, D)
    bv = ln_b.reshape(1, D)

    pools = []
    for k in range(3):
        wrel, wroot, bias = ((W_rel1, W_root1, b1) if k == 0
                             else (W_rel2, W_root2, b2))
        agg_lo, agg_hi = _sc_segsum(h_lo, h_hi, s0, d0, s1, d1, counts, zeros)
        h_lo, h_hi = _tc_layer(agg_lo, agg_hi, h_lo, h_hi,
                               wrel, wroot, bias.reshape(1, D), gv, bv)
        pools.append(_tc_pool(starts, h_lo, h_hi))

    hk = jnp.concatenate(pools, axis=1)
    w3p = jnp.pad(W_l3, ((0, 0), (0, 128 - W_l3.shape[1])))
    b3p = jnp.pad(b_l3, (0, 128 - b_l3.shape[0])).reshape(1, 128)
    out = _tc_head(hk, bn_g, bn_b, W_l1, b_l1, W_l2, b_l2, w3p, b3p)
    return out[:, :W_l3.shape[1]]


# speedup vs baseline: 3.6517x; 1.0072x over previous
import functools

import jax
import jax.numpy as jnp
from jax import lax
from jax.experimental import pallas as pl
from jax.experimental.pallas import tpu as pltpu
from jax.experimental.pallas import tpu_sc as plsc

N = 10000
E = 160000
D = 256
DH = 128
NG = 64
EPS = 1e-5
NSUB = 16
NW = 2 * NSUB
CH = 128
NP = 10240
NH = NP // 2
ACCR = NH + 128
ZPT = ACCR // NSUB
RPT = NH // NSUB
EPW = E // NW
NGRP = EPW // 16
BROW = 40
BCAP = BROW * CH


def _splat(val):
    return jnp.zeros((16,), jnp.int32) + val


def _gather16(x, idx):
    dnums = lax.GatherDimensionNumbers(
        offset_dims=(), collapsed_slice_dims=(0,), start_index_map=(0,))
    return lax.gather(x, idx[:, None], dnums, slice_sizes=(1,),
                      mode=lax.GatherScatterMode.PROMISE_IN_BOUNDS)


def _cumsum16(x):
    lane = lax.broadcasted_iota(jnp.int32, (16,), 0)
    for k in (1, 2, 4, 8):
        sh = _gather16(x, jnp.maximum(lane - k, 0))
        x = x + jnp.where(lane >= k, sh, 0)
    return x


def _compact16(m, vals):
    lane = lax.broadcasted_iota(jnp.int32, (16,), 0)
    cum = _cumsum16(jnp.where(m, 1, 0))
    c = cum[15]
    i1 = lane + 1
    pos = jnp.zeros((16,), jnp.int32)
    for s in (8, 4, 2, 1):
        nxt = pos + s
        cmid = _gather16(cum, nxt - 1)
        pos = jnp.where(cmid < i1, nxt, pos)
    pos = jnp.minimum(pos, 15)
    outm = lane < c
    return c, outm, [_gather16(v, pos) for v in vals]


def _route_body(src, dst, zeros_i, trash_i, s0, d0, s1, d1, counts,
                srcin, dstin, bs0, bd0, bs1, bd1, cbuf):
    cid = lax.axis_index("c")
    sid = lax.axis_index("s")
    wid = cid * NSUB + sid
    base = wid * EPW

    pltpu.sync_copy(src.at[pl.ds(base, EPW)], srcin)
    pltpu.sync_copy(dst.at[pl.ds(base, EPW)], dstin)
    pltpu.sync_copy(zeros_i, bs0)
    pltpu.sync_copy(zeros_i, bs1)
    pltpu.sync_copy(trash_i, bd0)
    pltpu.sync_copy(trash_i, bd1)

    def group(sv, dv, valid, off0, off1):
        if valid is None:
            m0 = dv < NH
            m1 = dv >= NH
        else:
            m0 = valid & (dv < NH)
            m1 = valid & (dv >= NH)
        c0, outm0, (sv0, dv0) = _compact16(m0, [sv, dv])
        bs0[pl.ds(off0, 16)] = jnp.where(outm0, sv0, 0)
        bd0[pl.ds(off0, 16)] = jnp.where(outm0, dv0, NH)
        c1, outm1, (sv1, dv1) = _compact16(m1, [sv, dv])
        bs1[pl.ds(off1, 16)] = jnp.where(outm1, sv1, 0)
        bd1[pl.ds(off1, 16)] = jnp.where(outm1, dv1 - NH, NH)
        return off0 + c0, off1 + c1

    def body(g, carry):
        off0, off1 = carry
        sv = srcin[pl.ds(g * 16, 16)]
        dv = dstin[pl.ds(g * 16, 16)]
        return group(sv, dv, None, off0, off1)

    off0, off1 = lax.fori_loop(0, NGRP, body, (0, 0))

    sv = srcin[pl.ds(EPW - 16, 16)]
    dv = dstin[pl.ds(EPW - 16, 16)]
    lane = lax.broadcasted_iota(jnp.int32, (16,), 0)
    off0, off1 = group(sv, dv, lane >= 8, off0, off1)

    for k in range(8):
        cbuf[0, pl.ds(k * 16, 16)] = _splat(off0)
        cbuf[1, pl.ds(k * 16, 16)] = _splat(off1)

    out0 = wid * BCAP
    pltpu.sync_copy(bs0, s0.at[pl.ds(out0, BCAP)])
    pltpu.sync_copy(bd0, d0.at[pl.ds(out0, BCAP)])
    pltpu.sync_copy(bs1, s1.at[pl.ds(out0, BCAP)])
    pltpu.sync_copy(bd1, d1.at[pl.ds(out0, BCAP)])
    pltpu.sync_copy(cbuf, counts.at[wid])


def _sc_route(src, dst, zeros_i, trash_i):
    mesh = plsc.VectorSubcoreMesh(core_axis_name="c", subcore_axis_name="s")
    bkt = jax.ShapeDtypeStruct((NW * BCAP,), jnp.int32)
    f = pl.kernel(
        _route_body,
        out_type=[bkt, bkt, bkt, bkt,
                  jax.ShapeDtypeStruct((NW, 8, CH), jnp.int32)],
        mesh=mesh,
        scratch_types=[
            pltpu.VMEM((EPW,), jnp.int32),
            pltpu.VMEM((EPW,), jnp.int32),
            pltpu.VMEM((BCAP,), jnp.int32),
            pltpu.VMEM((BCAP,), jnp.int32),
            pltpu.VMEM((BCAP,), jnp.int32),
            pltpu.VMEM((BCAP,), jnp.int32),
            pltpu.VMEM((8, CH), jnp.int32),
        ],
    )
    return f(src, dst, zeros_i, trash_i)


def _sc_body(h_lo, h_hi, s0, d0, s1, d1, counts, zeros, agg_lo, agg_hi,
             acc, vsrc, vdst, idx_s0, idx_d0, idx_s1, idx_d1, rows0, rows1,
             cbuf, buf, sem, sem2):
    cid = lax.axis_index("c")
    sid = lax.axis_index("s")
    idx_s = (idx_s0, idx_s1)
    idx_d = (idx_d0, idx_d1)
    rows = (rows0, rows1)

    def fill_idx(j, jb):
        for g in range(CH // 16):
            idx_s[jb][pl.ds(g * 16, 16)] = vsrc[pl.ds(j * CH + g * 16, 16)]
            idx_d[jb][pl.ds(g * 16, 16)] = vdst[pl.ds(j * CH + g * 16, 16)]

    def run_row(h_ref, nch):
        @pl.when(nch > 0)
        def _():
            fill_idx(0, 0)
            pltpu.async_copy(h_ref.at[idx_s[0]], rows[0], sem)

        def step(j, jb):
            pltpu.make_async_copy(h_ref.at[idx_s[jb]], rows[jb], sem).wait()

            @pl.when(j + 1 < nch)
            def _():
                @pl.when(j > 0)
                def _():
                    pltpu.make_async_copy(
                        rows[1 - jb], acc.at[idx_d[1 - jb]], sem2).wait()

                fill_idx(j + 1, 1 - jb)
                pltpu.async_copy(h_ref.at[idx_s[1 - jb]], rows[1 - jb], sem)

            pltpu.async_copy(rows[jb], acc.at[idx_d[jb]], sem2, add=True)

        def body(j, _):
            @pl.when(j % 2 == 0)
            def _():
                step(j, 0)

            @pl.when(j % 2 == 1)
            def _():
                step(j, 1)

            return 0

        lax.fori_loop(0, nch, body, 0)

        @pl.when(nch > 1)
        def _():
            @pl.when(nch % 2 == 0)
            def _():
                pltpu.make_async_copy(rows[0], acc.at[idx_d[0]], sem2).wait()

            @pl.when(nch % 2 == 1)
            def _():
                pltpu.make_async_copy(rows[1], acc.at[idx_d[1]], sem2).wait()

        @pl.when(nch > 0)
        def _():
            @pl.when(nch % 2 == 0)
            def _():
                pltpu.make_async_copy(rows[1], acc.at[idx_d[1]], sem2).wait()

            @pl.when(nch % 2 == 1)
            def _():
                pltpu.make_async_copy(rows[0], acc.at[idx_d[0]], sem2).wait()

    for p, (sb, db) in enumerate(((s0, d0), (s1, d1))):
        lo = p * NH

        pltpu.sync_copy(zeros, buf)
        pltpu.sync_copy(buf, acc.at[pl.ds(sid * ZPT, ZPT)])
        plsc.subcore_barrier()

        for rbo in range(2):
            rb = 2 * sid + rbo
            pltpu.sync_copy(sb.at[pl.ds(rb * BCAP, BCAP)], vsrc)
            pltpu.sync_copy(db.at[pl.ds(rb * BCAP, BCAP)], vdst)
            pltpu.sync_copy(counts.at[rb], cbuf)
            cnt = cbuf[p, pl.ds(0, 16)][0]
            nch = (cnt + CH - 1) // CH

            @pl.when(cid == 0)
            def _():
                run_row(h_lo, nch)

            @pl.when(cid == 1)
            def _():
                run_row(h_hi, nch)

        plsc.subcore_barrier()

        pltpu.sync_copy(acc.at[pl.ds(sid * RPT, RPT)], buf.at[pl.ds(0, RPT)])

        @pl.when(cid == 0)
        def _():
            pltpu.sync_copy(buf.at[pl.ds(0, RPT)],
                            agg_lo.at[pl.ds(lo + sid * RPT, RPT)])

        @pl.when(cid == 1)
        def _():
            pltpu.sync_copy(buf.at[pl.ds(0, RPT)],
                            agg_hi.at[pl.ds(lo + sid * RPT, RPT)])

        plsc.subcore_barrier()


def _sc_segsum(h_lo, h_hi, s0, d0, s1, d1, counts, zeros):
    mesh = plsc.VectorSubcoreMesh(core_axis_name="c", subcore_axis_name="s")
    f = pl.kernel(
        _sc_body,
        out_type=[jax.ShapeDtypeStruct((NP, DH), jnp.float32),
                  jax.ShapeDtypeStruct((NP, DH), jnp.float32)],
        mesh=mesh,
        scratch_types=[
            pltpu.VMEM_SHARED((ACCR, DH), jnp.float32),
            pltpu.VMEM((BCAP,), jnp.int32),
            pltpu.VMEM((BCAP,), jnp.int32),
            pltpu.VMEM((CH,), jnp.int32),
            pltpu.VMEM((CH,), jnp.int32),
            pltpu.VMEM((CH,), jnp.int32),
            pltpu.VMEM((CH,), jnp.int32),
            pltpu.VMEM((CH, DH), jnp.float32),
            pltpu.VMEM((CH, DH), jnp.float32),
            pltpu.VMEM((8, CH), jnp.int32),
            pltpu.VMEM((ZPT, DH), jnp.float32),
            pltpu.SemaphoreType.DMA,
            pltpu.SemaphoreType.DMA,
        ],
    )
    return f(h_lo, h_hi, s0, d0, s1, d1, counts, zeros)


_RB = 1000


def _layer_body(agg_lo, agg_hi, h_lo, h_hi, wrel, wroot, bias, g, b,
                out_lo, out_hi):
    t = jnp.dot(agg_lo[...], wrel[:DH, :], preferred_element_type=jnp.float32)
    t += jnp.dot(agg_hi[...], wrel[DH:, :], preferred_element_type=jnp.float32)
    t += jnp.dot(h_lo[...], wroot[:DH, :], preferred_element_type=jnp.float32)
    t += jnp.dot(h_hi[...], wroot[DH:, :], preferred_element_type=jnp.float32)
    t += bias[...]
    m = jnp.mean(t, axis=1, keepdims=True)
    d = t - m
    v = jnp.mean(d * d, axis=1, keepdims=True)
    y = d / jnp.sqrt(v + EPS) * g[...] + b[...]
    out_lo[...] = y[:, :DH]
    out_hi[...] = y[:, DH:]


def _tc_layer(agg_lo, agg_hi, h_lo, h_hi, wrel, wroot, bias, g, b):
    grid = (N // _RB,)
    row = pl.BlockSpec((_RB, DH), lambda i: (i, 0))
    full = pl.BlockSpec((D, D), lambda i: (0, 0))
    vec = pl.BlockSpec((1, D), lambda i: (0, 0))
    return pl.pallas_call(
        _layer_body,
        grid=grid,
        in_specs=[row, row, row, row, full, full, vec, vec, vec],
        out_specs=[row, row],
        out_shape=[jax.ShapeDtypeStruct((N, DH), jnp.float32),
                   jax.ShapeDtypeStruct((N, DH), jnp.float32)],
    )(agg_lo, agg_hi, h_lo, h_hi, wrel, wroot, bias, g, b)


_PC = 16


def _pool_body(starts, h_lo, h_hi, out):
    gidx = pl.program_id(0)
    s = starts[gidx]
    e = starts[gidx + 1]
    a0 = (s // _PC) * _PC
    nt = (e - a0 + _PC - 1) // _PC

    neg = jnp.float32(-3.0e38)
    init = (jnp.zeros((_PC, DH), jnp.float32), jnp.zeros((_PC, DH), jnp.float32),
            jnp.full((_PC, DH), neg), jnp.full((_PC, DH), neg))

    def body(t, carry):
        s_lo, s_hi, m_lo, m_hi = carry
        r0 = a0 + t * _PC
        rl = h_lo[pl.ds(r0, _PC), :]
        rh = h_hi[pl.ds(r0, _PC), :]
        ridx = r0 + lax.broadcasted_iota(jnp.int32, (_PC, 1), 0)
        mask = (ridx >= s) & (ridx < e)
        s_lo = s_lo + jnp.where(mask, rl, 0.0)
        s_hi = s_hi + jnp.where(mask, rh, 0.0)
        m_lo = jnp.maximum(m_lo, jnp.where(mask, rl, neg))
        m_hi = jnp.maximum(m_hi, jnp.where(mask, rh, neg))
        return s_lo, s_hi, m_lo, m_hi

    s_lo, s_hi, m_lo, m_hi = lax.fori_loop(0, nt, body, init)

    cnt = (e - s).astype(jnp.float32)
    has = cnt > 0.0
    inv = 1.0 / jnp.maximum(cnt, 1.0)
    sum_l = jnp.sum(s_lo, axis=0, keepdims=True)
    sum_h = jnp.sum(s_hi, axis=0, keepdims=True)
    max_l = jnp.max(m_lo, axis=0, keepdims=True)
    max_h = jnp.max(m_hi, axis=0, keepdims=True)
    max_l = jnp.where(has, max_l, 0.0)
    max_h = jnp.where(has, max_h, 0.0)
    out[0, :, 0:DH] = sum_l * inv
    out[0, :, DH:D] = sum_h * inv
    out[0, :, D:D + DH] = max_l
    out[0, :, D + DH:2 * D] = max_h
    out[0, :, 2 * D:2 * D + DH] = sum_l
    out[0, :, 2 * D + DH:3 * D] = sum_h


def _tc_pool(starts, h_lo, h_hi):
    grid_spec = pltpu.PrefetchScalarGridSpec(
        num_scalar_prefetch=1,
        grid=(NG,),
        in_specs=[pl.BlockSpec((N, DH), lambda i, st: (0, 0)),
                  pl.BlockSpec((N, DH), lambda i, st: (0, 0))],
        out_specs=pl.BlockSpec((1, 1, 3 * D), lambda i, st: (i, 0, 0)),
    )
    res = pl.pallas_call(
        _pool_body,
        grid_spec=grid_spec,
        out_shape=jax.ShapeDtypeStruct((NG, 1, 3 * D), jnp.float32),
    )(starts, h_lo, h_hi)
    return res.reshape(NG, 3 * D)


def _head_body(hk, bn_g, bn_b, w1, b1, w2, b2, w3, b3, out):
    h = hk[...]
    bm = jnp.mean(h, axis=0, keepdims=True)
    d = h - bm
    bv = jnp.mean(d * d, axis=0, keepdims=True)
    xn = d / jnp.sqrt(bv + EPS) * bn_g[...] + bn_b[...]
    x1 = jnp.maximum(jnp.dot(xn, w1[...], preferred_element_type=jnp.float32)
                     + b1[...], 0.0)
    x2 = jnp.maximum(jnp.dot(x1, w2[...], preferred_element_type=jnp.float32)
                     + b2[...], 0.0)
    lg = jnp.dot(x2, w3[...], preferred_element_type=jnp.float32) + b3[...]
    col = lax.broadcasted_iota(jnp.int32, lg.shape, 1)
    valid = col < 2
    lgm = jnp.where(valid, lg, jnp.float32(-3.0e38))
    mx = jnp.max(lgm, axis=1, keepdims=True)
    ls = lgm - mx
    se = jnp.sum(jnp.where(valid, jnp.exp(ls), 0.0), axis=1, keepdims=True)
    out[...] = ls - jnp.log(se)


def _tc_head(hk, bn_g, bn_b, w1, b1, w2, b2, w3p, b3p):
    dcat = 3 * D * 3
    return pl.pallas_call(
        _head_body,
        out_shape=jax.ShapeDtypeStruct((NG, 128), jnp.float32),
    )(hk, bn_g.reshape(1, dcat), bn_b.reshape(1, dcat),
      w1, b1.reshape(1, -1), w2, b2.reshape(1, -1), w3p, b3p)


def kernel(x, edge_index, batch, W_rel1, W_root1, b1, W_rel2, W_root2, b2,
           ln_g, ln_b, bn_g, bn_b, W_l1, b_l1, W_l2, b_l2, W_l3, b_l3):
    src = edge_index[0]
    dst = edge_index[1]
    starts = jnp.searchsorted(
        batch, jnp.arange(NG + 1, dtype=jnp.int32)).astype(jnp.int32)
    zeros = jnp.zeros((ZPT, DH), jnp.float32)
    zeros_i = jnp.zeros((BCAP,), jnp.int32)
    trash_i = jnp.zeros((BCAP,), jnp.int32) + NH

    s0, d0, s1, d1, counts = _sc_route(src, dst, zeros_i, trash_i)

    h_lo = x[:, :DH]
    h_hi = x[:, DH:]
    gv = ln_g.reshape(1, D)
    bv = ln_b.reshape(1, D)

    pools = []
    for k in range(3):
        wrel, wroot, bias = ((W_rel1, W_root1, b1) if k == 0
                             else (W_rel2, W_root2, b2))
        agg_lo, agg_hi = _sc_segsum(h_lo, h_hi, s0, d0, s1, d1, counts, zeros)
        h_lo, h_hi = _tc_layer(agg_lo, agg_hi, h_lo, h_hi,
                               wrel, wroot, bias.reshape(1, D), gv, bv)
        pools.append(_tc_pool(starts, h_lo, h_hi))

    hk = jnp.concatenate(pools, axis=1)
    w3p = jnp.pad(W_l3, ((0, 0), (0, 128 - W_l3.shape[1])))
    b3p = jnp.pad(b_l3, (0, 128 - b_l3.shape[0])).reshape(1, 128)
    out = _tc_head(hk, bn_g, bn_b, W_l1, b_l1, W_l2, b_l2, w3p, b3p)
    return out[:, :W_l3.shape[1]]
